# baseline (device time: 79833 ns/iter reference)
import itertools

import jax
import jax.numpy as jnp
from jax import lax
from jax.experimental import pallas as pl
from jax.experimental.pallas import tpu as pltpu

N_DEV = 32
CUBE = 8
NCUBE = 4
BLK = 128
CW = 128

_ORDERS = {
    "A": ("z", "y", "x"),
    "B": ("y", "x", "z"),
    "C": ("x", "z", "y"),
}
_PCS = [
    ("A", 0), ("B", 384), ("C", 768),
    ("A", 128), ("B", 512), ("C", 896),
    ("A", 256), ("B", 640),
]
NPC = len(_PCS)
_W = {"z": 512, "y": 256, "x": 128}


def _logical_id(q, p):
    z = 2 * (q // 2) + p // 4
    j = 4 * (q % 2) + p % 4
    return 8 * z + j


def kernel(x, w_mat):
    m, _ = x.shape
    _, n = w_mat.shape
    r2 = BLK // NCUBE

    def body(x_ref, w_ref, out_ref, acc_ref, rcv_ref, g2_ref,
             ss_rs, rs_rs, ss_ag, rs_ag, ss_l2, rs_l2, ss_l2b, rs_l2b):
        my = lax.axis_index("i")
        j = my % 8
        zplane = my // 8
        q = (j // 4) % 2 + 2 * (zplane // 2)
        p = j % 4 + 4 * (zplane % 2)
        bits = {"z": p // 4, "y": (p % 4) // 2, "x": p % 2}
        partner = {d: _logical_id(q, p + (1 - 2 * bits[d]) * (_W[d] // BLK))
                   for d in ("z", "y", "x")}
        my_row = p * BLK + q * r2
        group_peers = [(q + o) % NCUBE for o in range(1, NCUBE)]

        bar = pltpu.get_barrier_semaphore()
        for d in ("z", "y", "x"):
            pl.semaphore_signal(bar, inc=1, device_id=(partner[d],),
                                device_id_type=pl.DeviceIdType.MESH)
        for qq in group_peers:
            pl.semaphore_signal(bar, inc=1, device_id=(_logical_id(qq, p),),
                                device_id_type=pl.DeviceIdType.MESH)
        pl.semaphore_wait(bar, 6)

        acc_ref[...] = jnp.dot(
            x_ref[...], w_ref[...], preferred_element_type=jnp.float32
        )

        def blocks(order, stage, flip_to_partner):
            fixed, flip, free = order[:stage], order[stage], order[stage + 1:]
            base = sum(bits[d] * _W[d] for d in fixed)
            fb = (1 - bits[flip]) if flip_to_partner else bits[flip]
            base = base + fb * _W[flip]
            out = []
            slot0 = {0: 0, 1: 4, 2: 6}[stage]
            for si, combo in enumerate(itertools.product((0, 1), repeat=len(free))):
                out.append((base + sum(v * _W[d] for v, d in zip(combo, free)),
                            slot0 + si))
            return out

        def ag_blocks(order, stage):
            fixed, free = order[:3 - stage], order[3 - stage:]
            base = sum(bits[d] * _W[d] for d in fixed)
            out = []
            slot0 = {0: 0, 1: 1, 2: 3}[stage]
            for si, combo in enumerate(itertools.product((0, 1), repeat=len(free))):
                out.append((base + sum(v * _W[d] for v, d in zip(combo, free)),
                            slot0 + si))
            return out

        rs = [[None] * 7 for _ in range(NPC)]
        ag = [[None] * 7 for _ in range(NPC)]
        l2 = [[None] * (NCUBE - 1) for _ in range(NPC)]
        l2b = [[None] * (NCUBE - 1) for _ in range(NPC)]

        def col(c0):
            return pl.ds(c0, CW)

        def rs_stage_send(pc, stage):
            order, c0 = _ORDERS[_PCS[pc][0]], _PCS[pc][1]
            tgt = partner[order[stage]]
            for off, slot in blocks(order, stage, True):
                rdma = pltpu.make_async_remote_copy(
                    src_ref=acc_ref.at[pl.ds(off, BLK), col(c0)],
                    dst_ref=rcv_ref.at[pc, slot],
                    send_sem=ss_rs.at[pc, slot],
                    recv_sem=rs_rs.at[pc, slot],
                    device_id=(tgt,),
                    device_id_type=pl.DeviceIdType.MESH,
                )
                rdma.start()
                rs[pc][slot] = rdma

        def rs_stage_reduce(pc, stage):
            order, c0 = _ORDERS[_PCS[pc][0]], _PCS[pc][1]
            for off, slot in blocks(order, stage, False):
                rs[pc][slot].wait_recv()
                acc_ref[pl.ds(off, BLK), col(c0)] = (
                    acc_ref[pl.ds(off, BLK), col(c0)] + rcv_ref[pc, slot]
                )

        def ag_stage_send(pc, stage):
            order, c0 = _ORDERS[_PCS[pc][0]], _PCS[pc][1]
            tgt = partner[order[2 - stage]]
            for off, slot in ag_blocks(order, stage):
                rdma = pltpu.make_async_remote_copy(
                    src_ref=out_ref.at[pl.ds(off, BLK), col(c0)],
                    dst_ref=out_ref.at[pl.ds(off, BLK), col(c0)],
                    send_sem=ss_ag.at[pc, slot],
                    recv_sem=rs_ag.at[pc, slot],
                    device_id=(tgt,),
                    device_id_type=pl.DeviceIdType.MESH,
                )
                rdma.start()
                ag[pc][slot] = rdma

        def ag_stage_wait(pc, stage):
            for _, slot in ag_blocks(order=_ORDERS[_PCS[pc][0]], stage=stage):
                ag[pc][slot].wait_recv()

        for pc in range(NPC):
            rs_stage_send(pc, 0)
        for pc in range(NPC):
            rs_stage_reduce(pc, 0)
            rs_stage_send(pc, 1)
        for pc in range(NPC):
            rs_stage_reduce(pc, 1)
            rs_stage_send(pc, 2)

        for pc in range(NPC):
            rs_stage_reduce(pc, 2)
            c0 = _PCS[pc][1]
            for oi, qq in enumerate(group_peers):
                rdma = pltpu.make_async_remote_copy(
                    src_ref=acc_ref.at[pl.ds(p * BLK + qq * r2, r2), col(c0)],
                    dst_ref=g2_ref.at[pc, oi],
                    send_sem=ss_l2.at[pc, oi],
                    recv_sem=rs_l2.at[pc, oi],
                    device_id=(_logical_id(qq, p),),
                    device_id_type=pl.DeviceIdType.MESH,
                )
                rdma.start()
                l2[pc][oi] = rdma

        for pc in range(NPC):
            c0 = _PCS[pc][1]
            for rdma in l2[pc]:
                rdma.wait_recv()
            final = acc_ref[pl.ds(my_row, r2), col(c0)] + jnp.sum(
                g2_ref[pc], axis=0
            )
            out_ref[pl.ds(my_row, r2), col(c0)] = jnp.maximum(final, 0.0)
            for oi, qq in enumerate(group_peers):
                rdma = pltpu.make_async_remote_copy(
                    src_ref=out_ref.at[pl.ds(my_row, r2), col(c0)],
                    dst_ref=out_ref.at[pl.ds(my_row, r2), col(c0)],
                    send_sem=ss_l2b.at[pc, oi],
                    recv_sem=rs_l2b.at[pc, oi],
                    device_id=(_logical_id(qq, p),),
                    device_id_type=pl.DeviceIdType.MESH,
                )
                rdma.start()
                l2b[pc][oi] = rdma

        for pc in range(NPC):
            for rdma in l2b[pc]:
                rdma.wait_recv()
            ag_stage_send(pc, 0)
        for pc in range(NPC):
            ag_stage_wait(pc, 0)
            ag_stage_send(pc, 1)
        for pc in range(NPC):
            ag_stage_wait(pc, 1)
            ag_stage_send(pc, 2)
        for pc in range(NPC):
            ag_stage_wait(pc, 2)

        for pc in range(NPC):
            for rdma in rs[pc] + ag[pc] + l2[pc] + l2b[pc]:
                rdma.wait_send()

    return pl.pallas_call(
        body,
        out_shape=jax.ShapeDtypeStruct((m, n), jnp.float32),
        in_specs=[
            pl.BlockSpec(memory_space=pltpu.VMEM),
            pl.BlockSpec(memory_space=pltpu.VMEM),
        ],
        out_specs=pl.BlockSpec(memory_space=pltpu.VMEM),
        scratch_shapes=[
            pltpu.VMEM((m, n), jnp.float32),
            pltpu.VMEM((NPC, 7, BLK, CW), jnp.float32),
            pltpu.VMEM((NPC, NCUBE - 1, r2, CW), jnp.float32),
            pltpu.SemaphoreType.DMA((NPC, 7)),
            pltpu.SemaphoreType.DMA((NPC, 7)),
            pltpu.SemaphoreType.DMA((NPC, 7)),
            pltpu.SemaphoreType.DMA((NPC, 7)),
            pltpu.SemaphoreType.DMA((NPC, NCUBE - 1)),
            pltpu.SemaphoreType.DMA((NPC, NCUBE - 1)),
            pltpu.SemaphoreType.DMA((NPC, NCUBE - 1)),
            pltpu.SemaphoreType.DMA((NPC, NCUBE - 1)),
        ],
        compiler_params=pltpu.CompilerParams(collective_id=0),
    )(x, w_mat)


# device time: 43580 ns/iter; 1.8319x vs baseline; 1.8319x over previous
import jax
import jax.numpy as jnp
from jax import lax
from jax.experimental import pallas as pl
from jax.experimental.pallas import tpu as pltpu

N_DEV = 32
CUBE = 8
NCUBE = 4
NCHUNK = 8


def _logical_id(q, p):
    z = 2 * (q // 2) + p // 4
    j = 4 * (q % 2) + p % 4
    return 8 * z + j


def kernel(x, w_mat):
    m, _ = x.shape
    _, n = w_mat.shape
    r1 = m // CUBE
    r2 = r1 // NCUBE
    cw = n // NCHUNK

    def body(x_ref, w_ref, out_ref, acc_ref, acc16_ref, red_ref, out16_ref,
             g1_ref, g2_ref, ss1, ss2, ss3, ss4, rs1, rs2, rs3, rs4):
        my = lax.axis_index("i")
        j = my % 8
        zplane = my // 8
        q = (j // 4) % 2 + 2 * (zplane // 2)
        p = j % 4 + 4 * (zplane % 2)
        my_row = p * r1 + q * r2

        cube_peers = [(p + o) % CUBE for o in range(1, CUBE)]
        group_peers = [(q + o) % NCUBE for o in range(1, NCUBE)]

        bar = pltpu.get_barrier_semaphore()
        for pp in cube_peers:
            pl.semaphore_signal(bar, inc=1, device_id=(_logical_id(q, pp),),
                                device_id_type=pl.DeviceIdType.MESH)
        for qq in group_peers:
            pl.semaphore_signal(bar, inc=1, device_id=(_logical_id(qq, p),),
                                device_id_type=pl.DeviceIdType.MESH)
        pl.semaphore_wait(bar, CUBE - 1 + NCUBE - 1)

        acc_ref[...] = jnp.dot(
            x_ref[...], w_ref[...], preferred_element_type=jnp.float32
        )

        def col(c):
            return pl.ds(c * cw, cw)

        l1 = [[None] * (CUBE - 1) for _ in range(NCHUNK)]
        for c in range(NCHUNK):
            acc16_ref[:, col(c)] = acc_ref[:, col(c)].astype(jnp.bfloat16)
            for oi, pp in enumerate(cube_peers):
                rdma = pltpu.make_async_remote_copy(
                    src_ref=acc16_ref.at[pl.ds(pp * r1, r1), col(c)],
                    dst_ref=g1_ref.at[oi, :, col(c)],
                    send_sem=ss1.at[c, oi],
                    recv_sem=rs1.at[c, oi],
                    device_id=(_logical_id(q, pp),),
                    device_id_type=pl.DeviceIdType.MESH,
                )
                rdma.start()
                l1[c][oi] = rdma

        l2 = [[None] * (NCUBE - 1) for _ in range(NCHUNK)]
        for c in range(NCHUNK):
            for rdma in l1[c]:
                rdma.wait_recv()
            red = acc_ref[pl.ds(p * r1, r1), col(c)] + jnp.sum(
                g1_ref[:, :, col(c)].astype(jnp.float32), axis=0
            )
            red_ref[:, col(c)] = red
            acc16_ref[pl.ds(p * r1, r1), col(c)] = red.astype(jnp.bfloat16)
            for oi, qq in enumerate(group_peers):
                rdma = pltpu.make_async_remote_copy(
                    src_ref=acc16_ref.at[pl.ds(p * r1 + qq * r2, r2), col(c)],
                    dst_ref=g2_ref.at[oi, :, col(c)],
                    send_sem=ss2.at[c, oi],
                    recv_sem=rs2.at[c, oi],
                    device_id=(_logical_id(qq, p),),
                    device_id_type=pl.DeviceIdType.MESH,
                )
                rdma.start()
                l2[c][oi] = rdma

        l2b = [[None] * (NCUBE - 1) for _ in range(NCHUNK)]
        for c in range(NCHUNK):
            for rdma in l2[c]:
                rdma.wait_recv()
            final = red_ref[pl.ds(q * r2, r2), col(c)] + jnp.sum(
                g2_ref[:, :, col(c)].astype(jnp.float32), axis=0
            )
            out16_ref[pl.ds(my_row, r2), col(c)] = jnp.maximum(
                final, 0.0
            ).astype(jnp.bfloat16)
            for oi, qq in enumerate(group_peers):
                rdma = pltpu.make_async_remote_copy(
                    src_ref=out16_ref.at[pl.ds(my_row, r2), col(c)],
                    dst_ref=out16_ref.at[pl.ds(my_row, r2), col(c)],
                    send_sem=ss3.at[c, oi],
                    recv_sem=rs3.at[c, oi],
                    device_id=(_logical_id(qq, p),),
                    device_id_type=pl.DeviceIdType.MESH,
                )
                rdma.start()
                l2b[c][oi] = rdma

        l1b = [[None] * (CUBE - 1) for _ in range(NCHUNK)]
        for c in range(NCHUNK):
            for rdma in l2b[c]:
                rdma.wait_recv()
            for oi, pp in enumerate(cube_peers):
                rdma = pltpu.make_async_remote_copy(
                    src_ref=out16_ref.at[pl.ds(p * r1, r1), col(c)],
                    dst_ref=out16_ref.at[pl.ds(p * r1, r1), col(c)],
                    send_sem=ss4.at[c, oi],
                    recv_sem=rs4.at[c, oi],
                    device_id=(_logical_id(q, pp),),
                    device_id_type=pl.DeviceIdType.MESH,
                )
                rdma.start()
                l1b[c][oi] = rdma

        for c in range(NCHUNK):
            for rdma in l1b[c]:
                rdma.wait_recv()
            out_ref[:, col(c)] = out16_ref[:, col(c)].astype(jnp.float32)
        for group in (l1, l2, l2b, l1b):
            for c in range(NCHUNK):
                for rdma in group[c]:
                    rdma.wait_send()

    return pl.pallas_call(
        body,
        out_shape=jax.ShapeDtypeStruct((m, n), jnp.float32),
        in_specs=[
            pl.BlockSpec(memory_space=pltpu.VMEM),
            pl.BlockSpec(memory_space=pltpu.VMEM),
        ],
        out_specs=pl.BlockSpec(memory_space=pltpu.VMEM),
        scratch_shapes=[
            pltpu.VMEM((m, n), jnp.float32),
            pltpu.VMEM((m, n), jnp.bfloat16),
            pltpu.VMEM((r1, n), jnp.float32),
            pltpu.VMEM((m, n), jnp.bfloat16),
            pltpu.VMEM((CUBE - 1, r1, n), jnp.bfloat16),
            pltpu.VMEM((NCUBE - 1, r2, n), jnp.bfloat16),
            pltpu.SemaphoreType.DMA((NCHUNK, CUBE - 1)),
            pltpu.SemaphoreType.DMA((NCHUNK, NCUBE - 1)),
            pltpu.SemaphoreType.DMA((NCHUNK, NCUBE - 1)),
            pltpu.SemaphoreType.DMA((NCHUNK, CUBE - 1)),
            pltpu.SemaphoreType.DMA((NCHUNK, CUBE - 1)),
            pltpu.SemaphoreType.DMA((NCHUNK, NCUBE - 1)),
            pltpu.SemaphoreType.DMA((NCHUNK, NCUBE - 1)),
            pltpu.SemaphoreType.DMA((NCHUNK, CUBE - 1)),
        ],
        compiler_params=pltpu.CompilerParams(collective_id=0),
    )(x, w_mat)


# device time: 42059 ns/iter; 1.8981x vs baseline; 1.0362x over previous
import jax
import jax.numpy as jnp
from jax import lax
from jax.experimental import pallas as pl
from jax.experimental.pallas import tpu as pltpu

N_DEV = 32
CUBE = 8
NCUBE = 4
NCHUNK = 8


def _logical_id(q, p):
    z = 2 * (q // 2) + p // 4
    j = 4 * (q % 2) + p % 4
    return 8 * z + j


def kernel(x, w_mat):
    m, _ = x.shape
    _, n = w_mat.shape
    r1 = m // CUBE
    cw = n // NCHUNK

    def body(x_ref, w_ref, out_ref, acc_ref, acc16_ref, red_ref, out16_ref,
             g1_ref, g2_ref, ss1, ss2, ss4, rs1, rs2, rs4):
        my = lax.axis_index("i")
        j = my % 8
        zplane = my // 8
        q = (j // 4) % 2 + 2 * (zplane // 2)
        p = j % 4 + 4 * (zplane % 2)

        cube_peers = [(p + o) % CUBE for o in range(1, CUBE)]
        group_peers = [(q + o) % NCUBE for o in range(1, NCUBE)]

        bar = pltpu.get_barrier_semaphore()
        for pp in cube_peers:
            pl.semaphore_signal(bar, inc=1, device_id=(_logical_id(q, pp),),
                                device_id_type=pl.DeviceIdType.MESH)
        for qq in group_peers:
            pl.semaphore_signal(bar, inc=1, device_id=(_logical_id(qq, p),),
                                device_id_type=pl.DeviceIdType.MESH)
        pl.semaphore_wait(bar, CUBE - 1 + NCUBE - 1)

        acc_ref[...] = jnp.dot(
            x_ref[...], w_ref[...], preferred_element_type=jnp.float32
        )

        def col(c):
            return pl.ds(c * cw, cw)

        l1 = [[None] * (CUBE - 1) for _ in range(NCHUNK)]
        for c in range(NCHUNK):
            acc16_ref[:, col(c)] = acc_ref[:, col(c)].astype(jnp.bfloat16)
            for oi, pp in enumerate(cube_peers):
                rdma = pltpu.make_async_remote_copy(
                    src_ref=acc16_ref.at[pl.ds(pp * r1, r1), col(c)],
                    dst_ref=g1_ref.at[oi, :, col(c)],
                    send_sem=ss1.at[c, oi],
                    recv_sem=rs1.at[c, oi],
                    device_id=(_logical_id(q, pp),),
                    device_id_type=pl.DeviceIdType.MESH,
                )
                rdma.start()
                l1[c][oi] = rdma

        l2 = [[None] * (NCUBE - 1) for _ in range(NCHUNK)]
        for c in range(NCHUNK):
            for rdma in l1[c]:
                rdma.wait_recv()
            red = acc_ref[pl.ds(p * r1, r1), col(c)] + jnp.sum(
                g1_ref[:, :, col(c)].astype(jnp.float32), axis=0
            )
            red_ref[:, col(c)] = red
            acc16_ref[pl.ds(p * r1, r1), col(c)] = red.astype(jnp.bfloat16)
            for oi, qq in enumerate(group_peers):
                rdma = pltpu.make_async_remote_copy(
                    src_ref=acc16_ref.at[pl.ds(p * r1, r1), col(c)],
                    dst_ref=g2_ref.at[oi, :, col(c)],
                    send_sem=ss2.at[c, oi],
                    recv_sem=rs2.at[c, oi],
                    device_id=(_logical_id(qq, p),),
                    device_id_type=pl.DeviceIdType.MESH,
                )
                rdma.start()
                l2[c][oi] = rdma

        l1b = [[None] * (CUBE - 1) for _ in range(NCHUNK)]
        for c in range(NCHUNK):
            for rdma in l2[c]:
                rdma.wait_recv()
            final = red_ref[:, col(c)] + jnp.sum(
                g2_ref[:, :, col(c)].astype(jnp.float32), axis=0
            )
            out16_ref[pl.ds(p * r1, r1), col(c)] = jnp.maximum(
                final, 0.0
            ).astype(jnp.bfloat16)
            for oi, pp in enumerate(cube_peers):
                rdma = pltpu.make_async_remote_copy(
                    src_ref=out16_ref.at[pl.ds(p * r1, r1), col(c)],
                    dst_ref=out16_ref.at[pl.ds(p * r1, r1), col(c)],
                    send_sem=ss4.at[c, oi],
                    recv_sem=rs4.at[c, oi],
                    device_id=(_logical_id(q, pp),),
                    device_id_type=pl.DeviceIdType.MESH,
                )
                rdma.start()
                l1b[c][oi] = rdma

        for c in range(NCHUNK):
            for rdma in l1b[c]:
                rdma.wait_recv()
            out_ref[:, col(c)] = out16_ref[:, col(c)].astype(jnp.float32)
        for group in (l1, l2, l1b):
            for c in range(NCHUNK):
                for rdma in group[c]:
                    rdma.wait_send()

    return pl.pallas_call(
        body,
        out_shape=jax.ShapeDtypeStruct((m, n), jnp.float32),
        in_specs=[
            pl.BlockSpec(memory_space=pltpu.VMEM),
            pl.BlockSpec(memory_space=pltpu.VMEM),
        ],
        out_specs=pl.BlockSpec(memory_space=pltpu.VMEM),
        scratch_shapes=[
            pltpu.VMEM((m, n), jnp.float32),
            pltpu.VMEM((m, n), jnp.bfloat16),
            pltpu.VMEM((r1, n), jnp.float32),
            pltpu.VMEM((m, n), jnp.bfloat16),
            pltpu.VMEM((CUBE - 1, r1, n), jnp.bfloat16),
            pltpu.VMEM((NCUBE - 1, r1, n), jnp.bfloat16),
            pltpu.SemaphoreType.DMA((NCHUNK, CUBE - 1)),
            pltpu.SemaphoreType.DMA((NCHUNK, NCUBE - 1)),
            pltpu.SemaphoreType.DMA((NCHUNK, CUBE - 1)),
            pltpu.SemaphoreType.DMA((NCHUNK, CUBE - 1)),
            pltpu.SemaphoreType.DMA((NCHUNK, NCUBE - 1)),
            pltpu.SemaphoreType.DMA((NCHUNK, CUBE - 1)),
        ],
        compiler_params=pltpu.CompilerParams(collective_id=0),
    )(x, w_mat)


# device time: 41104 ns/iter; 1.9422x vs baseline; 1.0232x over previous
import jax
import jax.numpy as jnp
from jax import lax
from jax.experimental import pallas as pl
from jax.experimental.pallas import tpu as pltpu

N_DEV = 32
CUBE = 8
NCUBE = 4
NCHUNK = 4


def _logical_id(q, p):
    z = 2 * (q // 2) + p // 4
    j = 4 * (q % 2) + p % 4
    return 8 * z + j


def kernel(x, w_mat):
    m, _ = x.shape
    _, n = w_mat.shape
    r1 = m // CUBE
    cw = n // NCHUNK

    def body(x_ref, w_ref, out_ref, acc_ref, acc16_ref, red_ref, out16_ref,
             g1_ref, g2_ref, ss1, ss2, ss4, rs1, rs2, rs4):
        my = lax.axis_index("i")
        j = my % 8
        zplane = my // 8
        q = (j // 4) % 2 + 2 * (zplane // 2)
        p = j % 4 + 4 * (zplane % 2)

        cube_peers = [(p + o) % CUBE for o in range(1, CUBE)]
        group_peers = [(q + o) % NCUBE for o in range(1, NCUBE)]

        bar = pltpu.get_barrier_semaphore()
        for pp in cube_peers:
            pl.semaphore_signal(bar, inc=1, device_id=(_logical_id(q, pp),),
                                device_id_type=pl.DeviceIdType.MESH)
        for qq in group_peers:
            pl.semaphore_signal(bar, inc=1, device_id=(_logical_id(qq, p),),
                                device_id_type=pl.DeviceIdType.MESH)
        pl.semaphore_wait(bar, CUBE - 1 + NCUBE - 1)

        acc_ref[...] = jnp.dot(
            x_ref[...], w_ref[...], preferred_element_type=jnp.float32
        )

        def col(c):
            return pl.ds(c * cw, cw)

        l1 = [[None] * (CUBE - 1) for _ in range(NCHUNK)]
        for c in range(NCHUNK):
            acc16_ref[:, col(c)] = acc_ref[:, col(c)].astype(jnp.bfloat16)
            for oi, pp in enumerate(cube_peers):
                rdma = pltpu.make_async_remote_copy(
                    src_ref=acc16_ref.at[pl.ds(pp * r1, r1), col(c)],
                    dst_ref=g1_ref.at[oi, :, col(c)],
                    send_sem=ss1.at[c, oi],
                    recv_sem=rs1.at[c, oi],
                    device_id=(_logical_id(q, pp),),
                    device_id_type=pl.DeviceIdType.MESH,
                )
                rdma.start()
                l1[c][oi] = rdma

        l2 = [[None] * (NCUBE - 1) for _ in range(NCHUNK)]
        for c in range(NCHUNK):
            for rdma in l1[c]:
                rdma.wait_recv()
            red = acc_ref[pl.ds(p * r1, r1), col(c)] + jnp.sum(
                g1_ref[:, :, col(c)].astype(jnp.float32), axis=0
            )
            red_ref[:, col(c)] = red
            acc16_ref[pl.ds(p * r1, r1), col(c)] = red.astype(jnp.bfloat16)
            for oi, qq in enumerate(group_peers):
                rdma = pltpu.make_async_remote_copy(
                    src_ref=acc16_ref.at[pl.ds(p * r1, r1), col(c)],
                    dst_ref=g2_ref.at[oi, :, col(c)],
                    send_sem=ss2.at[c, oi],
                    recv_sem=rs2.at[c, oi],
                    device_id=(_logical_id(qq, p),),
                    device_id_type=pl.DeviceIdType.MESH,
                )
                rdma.start()
                l2[c][oi] = rdma

        l1b = [[None] * (CUBE - 1) for _ in range(NCHUNK)]
        for c in range(NCHUNK):
            for rdma in l2[c]:
                rdma.wait_recv()
            final = red_ref[:, col(c)] + jnp.sum(
                g2_ref[:, :, col(c)].astype(jnp.float32), axis=0
            )
            out16_ref[pl.ds(p * r1, r1), col(c)] = jnp.maximum(
                final, 0.0
            ).astype(jnp.bfloat16)
            for oi, pp in enumerate(cube_peers):
                rdma = pltpu.make_async_remote_copy(
                    src_ref=out16_ref.at[pl.ds(p * r1, r1), col(c)],
                    dst_ref=out16_ref.at[pl.ds(p * r1, r1), col(c)],
                    send_sem=ss4.at[c, oi],
                    recv_sem=rs4.at[c, oi],
                    device_id=(_logical_id(q, pp),),
                    device_id_type=pl.DeviceIdType.MESH,
                )
                rdma.start()
                l1b[c][oi] = rdma

        for c in range(NCHUNK):
            for rdma in l1b[c]:
                rdma.wait_recv()
            out_ref[:, col(c)] = out16_ref[:, col(c)].astype(jnp.float32)
        for group in (l1, l2, l1b):
            for c in range(NCHUNK):
                for rdma in group[c]:
                    rdma.wait_send()

    return pl.pallas_call(
        body,
        out_shape=jax.ShapeDtypeStruct((m, n), jnp.float32),
        in_specs=[
            pl.BlockSpec(memory_space=pltpu.VMEM),
            pl.BlockSpec(memory_space=pltpu.VMEM),
        ],
        out_specs=pl.BlockSpec(memory_space=pltpu.VMEM),
        scratch_shapes=[
            pltpu.VMEM((m, n), jnp.float32),
            pltpu.VMEM((m, n), jnp.bfloat16),
            pltpu.VMEM((r1, n), jnp.float32),
            pltpu.VMEM((m, n), jnp.bfloat16),
            pltpu.VMEM((CUBE - 1, r1, n), jnp.bfloat16),
            pltpu.VMEM((NCUBE - 1, r1, n), jnp.bfloat16),
            pltpu.SemaphoreType.DMA((NCHUNK, CUBE - 1)),
            pltpu.SemaphoreType.DMA((NCHUNK, NCUBE - 1)),
            pltpu.SemaphoreType.DMA((NCHUNK, CUBE - 1)),
            pltpu.SemaphoreType.DMA((NCHUNK, CUBE - 1)),
            pltpu.SemaphoreType.DMA((NCHUNK, NCUBE - 1)),
            pltpu.SemaphoreType.DMA((NCHUNK, CUBE - 1)),
        ],
        compiler_params=pltpu.CompilerParams(collective_id=0),
    )(x, w_mat)
